# Initial kernel scaffold; baseline (speedup 1.0000x reference)
import jax
import jax.numpy as jnp
from jax.experimental import pallas as pl


def kernel(x, edge_index, edge_type, batch, W1_rel, W1_root, b1, g1, be1, W2_rel, W2_root, b2, g2, be2, W3_rel, W3_root, b3, g3, be3, Wf1, bf1, gf, bef, Wf2, bf2, Wd1, bd1, Wd2, bd2):
    raise NotImplementedError("write your pallas kernel here")


# speedup vs baseline: 10.7853x; 10.7853x over previous
import functools

import jax
import jax.numpy as jnp
from jax import lax
from jax.experimental import pallas as pl
from jax.experimental.pallas import tpu as pltpu
from jax.experimental.pallas import tpu_sc as plsc

N = 10000
E = 320000
D = 128
R = 8
H = 64
G = 128
EMB = 16
OUT = 8
EPS = 1e-5

NC = 2
NS = 16
NW = NC * NS
CH = 128
CPT = 80
E_PAD = NW * CPT * CH
NROWS = E_PAD // CH
NPT = N // NS
NPAD = N + 64

_mesh = plsc.VectorSubcoreMesh(
    core_axis_name="c", subcore_axis_name="s", num_cores=NC, num_subcores=NS)


def _wid():
    return lax.axis_index("s") * NC + lax.axis_index("c")


def _cnt_body(dst2d, et2d, zc, cnt_out, cnt_sh, dstb, etb, oh):
    c = lax.axis_index("c")
    s = lax.axis_index("s")
    wid = _wid()
    pltpu.sync_copy(zc.at[pl.ds(s * NPT, NPT)], cnt_sh.at[pl.ds(s * NPT, NPT)])
    plsc.subcore_barrier()

    ones16 = jnp.full((16,), 1.0, jnp.float32)
    zeros16 = jnp.zeros((16,), jnp.float32)

    def zrow(i, _):
        oh[i, :] = zeros16
        return 0

    lax.fori_loop(0, CH, zrow, 0)

    def chunk(j, _):
        row = wid * CPT + j
        pltpu.sync_copy(dst2d.at[row], dstb.at[0])
        pltpu.sync_copy(et2d.at[row], etb.at[0])
        idx_list = []
        for gi in range(CH // 16):
            rows16 = lax.iota(jnp.int32, 16) + gi * 16
            e16 = etb[0, pl.ds(gi * 16, 16)]
            plsc.store_scatter(oh, [rows16, e16], ones16)
            idx_list.append((rows16, e16))
        pltpu.sync_copy(oh, cnt_sh.at[dstb.at[0]], add=True)
        for rows16, e16 in idx_list:
            plsc.store_scatter(oh, [rows16, e16], zeros16)
        return 0

    lax.fori_loop(0, CPT, chunk, 0)
    plsc.subcore_barrier()
    pltpu.sync_copy(cnt_sh.at[pl.ds(s * NPT, NPT)],
                    cnt_out.at[c, pl.ds(s * NPT, NPT)])


_cnt_kernel = pl.kernel(
    _cnt_body,
    out_type=jax.ShapeDtypeStruct((NC, N, 16), jnp.float32),
    mesh=_mesh,
    compiler_params=pltpu.CompilerParams(use_tc_tiling_on_sc=False, needs_layout_passes=False),
    scratch_types=[
        pltpu.VMEM_SHARED((N, 16), jnp.float32),
        pltpu.VMEM((1, CH), jnp.int32),
        pltpu.VMEM((1, CH), jnp.int32),
        pltpu.VMEM((CH, 16), jnp.float32),
    ],
)


def _prep_tc_body(cnt_ref, batch_ref, inv_ref, offs_ref, cntg_ref):
    cnt = cnt_ref[0, :, :R] + cnt_ref[1, :, :R]
    inv_ref[...] = 1.0 / jnp.maximum(cnt, 1.0)
    b = batch_ref[...]
    gid = lax.broadcasted_iota(jnp.int32, (1, G), 1)
    onehot = (b[:, None] == gid).astype(jnp.float32)
    counts = jnp.sum(onehot, axis=0)
    tri = (lax.broadcasted_iota(jnp.int32, (G, G), 0)
           <= lax.broadcasted_iota(jnp.int32, (G, G), 1)).astype(jnp.float32)
    csum = jnp.dot(counts, tri, preferred_element_type=jnp.float32, precision=lax.Precision.HIGHEST)
    offs = jnp.concatenate(
        [jnp.zeros((1,), jnp.float32), csum,
         jnp.full((15,), float(N), jnp.float32)]).astype(jnp.int32)
    offs_ref[...] = offs
    invg = jnp.where(counts > 0, 1.0 / jnp.maximum(counts, 1.0), 0.0)
    cntg_ref[...] = jnp.concatenate([invg, jnp.zeros((16,), jnp.float32)])


_prep_tc = pl.pallas_call(
    _prep_tc_body,
    out_shape=(
        jax.ShapeDtypeStruct((N, R), jnp.float32),
        jax.ShapeDtypeStruct((G + 16, ), jnp.int32),
        jax.ShapeDtypeStruct((G + 16,), jnp.float32),
    ),
)


def _prep_w_body(src2d, dst2d, et2d, inv_hbm, gidx_out, w_out,
                 invbuf, srcb, dstb, etb, gidxb, wb):
    wid = _wid()
    pltpu.sync_copy(inv_hbm, invbuf)

    def chunk(j, _):
        row = wid * CPT + j
        pltpu.sync_copy(src2d.at[row], srcb.at[0])
        pltpu.sync_copy(dst2d.at[row], dstb.at[0])
        pltpu.sync_copy(et2d.at[row], etb.at[0])
        for gi in range(CH // 16):
            sl = pl.ds(gi * 16, 16)
            s16 = srcb[0, sl]
            d16 = dstb[0, sl]
            e16 = etb[0, sl]
            mask = e16 < R
            gidxb[0, sl] = jnp.where(mask, e16 * N + s16, 0)
            pidx = d16 * R + e16
            w16 = plsc.load_gather(invbuf, [pidx])
            wb[sl] = jnp.where(mask, w16, 0.0)
        pltpu.sync_copy(gidxb.at[0], gidx_out.at[row])
        pltpu.sync_copy(wb, w_out.at[row])
        return 0

    lax.fori_loop(0, CPT, chunk, 0)


_prep_w = pl.kernel(
    _prep_w_body,
    out_type=(
        jax.ShapeDtypeStruct((NROWS, CH), jnp.int32),
        jax.ShapeDtypeStruct((NROWS, CH), jnp.float32),
    ),
    mesh=_mesh,
    compiler_params=pltpu.CompilerParams(use_tc_tiling_on_sc=False, needs_layout_passes=False),
    scratch_types=[
        pltpu.VMEM((N * R,), jnp.float32),
        pltpu.VMEM((1, CH), jnp.int32),
        pltpu.VMEM((1, CH), jnp.int32),
        pltpu.VMEM((1, CH), jnp.int32),
        pltpu.VMEM((1, CH), jnp.int32),
        pltpu.VMEM((CH,), jnp.float32),
    ],
)


def _msg_body(h_hbm, gidx2d, w2d, dst2d, za, agg_out,
              acc_sh, rows, gidxb, dstb, wb, sem):
    c = lax.axis_index("c")
    s = lax.axis_index("s")
    wid = _wid()
    pltpu.sync_copy(za.at[pl.ds(s * NPT, NPT)], acc_sh.at[pl.ds(s * NPT, NPT)])
    plsc.subcore_barrier()

    def chunk(j, _):
        row = wid * CPT + j
        pltpu.sync_copy(gidx2d.at[row], gidxb.at[0])
        pltpu.sync_copy(w2d.at[row], wb)
        pltpu.sync_copy(dst2d.at[row], dstb.at[0])
        pltpu.async_copy(h_hbm.at[gidxb.at[0]], rows, sem).wait()

        def scale(i, _):
            wsp = plsc.load_gather(wb, [jnp.full((16,), i, jnp.int32)])
            for q in range(H // 16):
                sl = pl.ds(q * 16, 16)
                rows[i, sl] = rows[i, sl] * wsp
            return 0

        lax.fori_loop(0, CH, scale, 0)
        pltpu.sync_copy(rows, acc_sh.at[dstb.at[0]], add=True)
        return 0

    lax.fori_loop(0, CPT, chunk, 0)
    plsc.subcore_barrier()
    pltpu.sync_copy(acc_sh.at[pl.ds(s * NPT, NPT)],
                    agg_out.at[c, pl.ds(s * NPT, NPT)])


_msg_kernel = pl.kernel(
    _msg_body,
    out_type=jax.ShapeDtypeStruct((NC, N, H), jnp.float32),
    mesh=_mesh,
    compiler_params=pltpu.CompilerParams(use_tc_tiling_on_sc=False, needs_layout_passes=False),
    scratch_types=[
        pltpu.VMEM_SHARED((N, H), jnp.float32),
        pltpu.VMEM((CH, H), jnp.float32),
        pltpu.VMEM((1, CH), jnp.int32),
        pltpu.VMEM((1, CH), jnp.int32),
        pltpu.VMEM((CH,), jnp.float32),
        pltpu.SemaphoreType.DMA,
    ],
)


RT = 2000
NT = N // RT


def _h_body(x_ref, wrel_ref, wroot_ref, b_ref, h_ref, root_ref):
    r = pl.program_id(1)
    h_ref[0] = jnp.dot(x_ref[...], wrel_ref[0],
                       preferred_element_type=jnp.float32)

    @pl.when(r == 0)
    def _():
        root_ref[...] = (jnp.dot(x_ref[...], wroot_ref[...],
                                 preferred_element_type=jnp.float32)
                         + b_ref[...][None, :])


def _make_h_kernel(din):
    return pl.pallas_call(
        _h_body,
        grid=(NT, R),
        in_specs=[
            pl.BlockSpec((RT, din), lambda t, r: (t, 0)),
            pl.BlockSpec((1, din, H), lambda t, r: (r, 0, 0)),
            pl.BlockSpec((din, H), lambda t, r: (0, 0)),
            pl.BlockSpec((H,), lambda t, r: (0,)),
        ],
        out_specs=[
            pl.BlockSpec((1, RT, H), lambda t, r: (r, t, 0)),
            pl.BlockSpec((RT, H), lambda t, r: (t, 0)),
        ],
        out_shape=(
            jax.ShapeDtypeStruct((R, N, H), jnp.float32),
            jax.ShapeDtypeStruct((N, H), jnp.float32),
        ),
    )


_h_kernel_1 = _make_h_kernel(D)
_h_kernel_2 = _make_h_kernel(H)


def _bn_body(agg_ref, root_ref, g_ref, be_ref, out_ref):
    a = agg_ref[0] + agg_ref[1] + root_ref[...]
    m = jnp.mean(a, axis=0)
    v = jnp.mean((a - m[None, :]) ** 2, axis=0)
    y = (a - m[None, :]) / jnp.sqrt(v[None, :] + EPS) * g_ref[...][None, :] + be_ref[...][None, :]
    y = jnp.maximum(y, 0.0)
    out_ref[...] = jnp.concatenate(
        [y, jnp.zeros((NPAD - N, H), jnp.float32)], axis=0)


_bn_kernel = pl.pallas_call(
    _bn_body,
    out_shape=jax.ShapeDtypeStruct((NPAD, H), jnp.float32),
)


def _pool_body(x1, x2, x3, offs_hbm, cntg_hbm, out_hbm,
               offb, cntb, b1, b2, b3, outb, sem):
    wid = _wid()
    pltpu.sync_copy(offs_hbm, offb.at[0])
    pltpu.sync_copy(cntg_hbm, cntb)
    NQ = (3 * H) // 16

    for k in range(G // NW):
        g = wid * (G // NW) + k
        ovec = offb[0, pl.ds(g, 16)]
        s0 = ovec[0]
        e0 = ovec[1]
        nch = (e0 - s0 + 63) // 64

        def chunk(ci, carry):
            sums, maxs = carry
            base = s0 + ci * 64
            cp1 = pltpu.async_copy(x1.at[pl.ds(base, 64)], b1, sem)
            cp2 = pltpu.async_copy(x2.at[pl.ds(base, 64)], b2, sem)
            cp3 = pltpu.async_copy(x3.at[pl.ds(base, 64)], b3, sem)
            cp1.wait(); cp2.wait(); cp3.wait()

            def row(i, carry2):
                sums2, maxs2 = carry2
                vals = []
                for q in range(4):
                    vals.append(b1[i, pl.ds(q * 16, 16)])
                for q in range(4):
                    vals.append(b2[i, pl.ds(q * 16, 16)])
                for q in range(4):
                    vals.append(b3[i, pl.ds(q * 16, 16)])
                act = (base + i) < e0
                ns = tuple(jnp.where(act, s + v, s) for s, v in zip(sums2, vals))
                nm = tuple(jnp.where(act, jnp.maximum(m, v), m)
                           for m, v in zip(maxs2, vals))
                return (ns, nm)

            return lax.fori_loop(0, 64, row, (sums, maxs))

        zero16 = jnp.zeros((16,), jnp.float32)
        ninf16 = jnp.full((16,), -jnp.inf, jnp.float32)
        sums0 = tuple(zero16 for _ in range(NQ))
        maxs0 = tuple(ninf16 for _ in range(NQ))
        sums, maxs = lax.fori_loop(0, nch, chunk, (sums0, maxs0))

        scale = cntb[pl.ds(g, 16)][0]
        nonempty = scale > 0.0
        for q in range(NQ):
            outb[0, pl.ds(q * 16, 16)] = sums[q] * scale
            outb[0, pl.ds((NQ + q) * 16, 16)] = jnp.where(
                nonempty, maxs[q], jnp.zeros((16,), jnp.float32))
        pltpu.sync_copy(outb.at[0], out_hbm.at[g])


_pool_kernel = pl.kernel(
    _pool_body,
    out_type=jax.ShapeDtypeStruct((G, 6 * H), jnp.float32),
    mesh=_mesh,
    compiler_params=pltpu.CompilerParams(use_tc_tiling_on_sc=False, needs_layout_passes=False),
    scratch_types=[
        pltpu.VMEM((1, G + 16), jnp.int32),
        pltpu.VMEM((G + 16,), jnp.float32),
        pltpu.VMEM((64, H), jnp.float32),
        pltpu.VMEM((64, H), jnp.float32),
        pltpu.VMEM((64, H), jnp.float32),
        pltpu.VMEM((1, 6 * H), jnp.float32),
        pltpu.SemaphoreType.DMA,
    ],
)


def _head_body(pool_ref, wf1_ref, bf1_ref, gf_ref, bef_ref, wf2_ref, bf2_ref,
               wd1_ref, bd1_ref, wd2_ref, bd2_ref, pred_ref, emb_ref):
    xg = pool_ref[...]
    h = jnp.dot(xg, wf1_ref[...], preferred_element_type=jnp.float32) + bf1_ref[...][None, :]
    m = jnp.mean(h, axis=0)
    v = jnp.mean((h - m[None, :]) ** 2, axis=0)
    h = (h - m[None, :]) / jnp.sqrt(v[None, :] + EPS) * gf_ref[...][None, :] + bef_ref[...][None, :]
    h = jnp.where(h > 0, h, 0.2 * h)
    emb = jnp.dot(h, wf2_ref[...], preferred_element_type=jnp.float32) + bf2_ref[...][None, :]
    t = jnp.dot(emb, wd1_ref[...], preferred_element_type=jnp.float32) + bd1_ref[...][None, :]
    t = jnp.where(t > 0, t, 0.2 * t)
    pred_ref[...] = jnp.dot(t, wd2_ref[...], preferred_element_type=jnp.float32) + bd2_ref[...][None, :]
    emb_ref[...] = emb


_head_kernel = pl.pallas_call(
    _head_body,
    out_shape=(
        jax.ShapeDtypeStruct((G, OUT), jnp.float32),
        jax.ShapeDtypeStruct((G, EMB), jnp.float32),
    ),
)


def kernel(x, edge_index, edge_type, batch,
           W1_rel, W1_root, b1, g1, be1,
           W2_rel, W2_root, b2, g2, be2,
           W3_rel, W3_root, b3, g3, be3,
           Wf1, bf1, gf, bef, Wf2, bf2, Wd1, bd1, Wd2, bd2):
    src = edge_index[0]
    dst = edge_index[1]
    pad = E_PAD - E
    src2d = jnp.pad(src, (0, pad)).reshape(NROWS, CH)
    dst2d = jnp.pad(dst, (0, pad)).reshape(NROWS, CH)
    et2d = jnp.pad(edge_type, (0, pad), constant_values=15).reshape(NROWS, CH)
    zc = jnp.zeros((N, 16), jnp.float32)
    za = jnp.zeros((N, H), jnp.float32)

    cnt_part = _cnt_kernel(dst2d, et2d, zc)
    inv2, offs, cntg = _prep_tc(cnt_part, batch)
    gidx2d, w2d = _prep_w(src2d, dst2d, et2d, inv2.reshape(-1))

    def layer(h_kernel, xprev, W_rel, W_root, b, g, be):
        h, root = h_kernel(xprev[:N], W_rel, W_root, b)
        agg_part = _msg_kernel(h.reshape(R * N, H), gidx2d, w2d, dst2d, za)
        return _bn_kernel(agg_part, root, g, be)

    x1 = layer(_h_kernel_1, x, W1_rel, W1_root, b1, g1, be1)
    x2 = layer(_h_kernel_2, x1, W2_rel, W2_root, b2, g2, be2)
    x3 = layer(_h_kernel_2, x2, W3_rel, W3_root, b3, g3, be3)

    pooled = _pool_kernel(x1, x2, x3, offs, cntg)
    pred, emb = _head_kernel(pooled, Wf1, bf1, gf, bef, Wf2, bf2,
                             Wd1, bd1, Wd2, bd2)
    return (pred, emb)


# speedup vs baseline: 15.4808x; 1.4354x over previous
import functools

import jax
import jax.numpy as jnp
from jax import lax
from jax.experimental import pallas as pl
from jax.experimental.pallas import tpu as pltpu
from jax.experimental.pallas import tpu_sc as plsc

N = 10000
E = 320000
D = 128
R = 8
H = 64
G = 128
EMB = 16
OUT = 8
EPS = 1e-5

NC = 2
NS = 16
NW = NC * NS
CH = 128
CPT = 80
E_PAD = NW * CPT * CH
NROWS = E_PAD // CH
NPT = N // NS
NPAD = N + 64

_mesh = plsc.VectorSubcoreMesh(
    core_axis_name="c", subcore_axis_name="s", num_cores=NC, num_subcores=NS)


def _wid():
    return lax.axis_index("s") * NC + lax.axis_index("c")


def _cnt_body(dst2d, et2d, zc, cnt_out, cnt_sh, dstb, etb, oh):
    c = lax.axis_index("c")
    s = lax.axis_index("s")
    wid = _wid()
    pltpu.sync_copy(zc.at[pl.ds(s * NPT, NPT)], cnt_sh.at[pl.ds(s * NPT, NPT)])
    plsc.subcore_barrier()

    ones16 = jnp.full((16,), 1.0, jnp.float32)
    zeros16 = jnp.zeros((16,), jnp.float32)

    def zrow(i, _):
        oh[i, :] = zeros16
        return 0

    lax.fori_loop(0, CH, zrow, 0)

    def chunk(j, _):
        row = wid * CPT + j
        pltpu.sync_copy(dst2d.at[row], dstb.at[0])
        pltpu.sync_copy(et2d.at[row], etb.at[0])
        idx_list = []
        for gi in range(CH // 16):
            rows16 = lax.iota(jnp.int32, 16) + gi * 16
            e16 = etb[0, pl.ds(gi * 16, 16)]
            plsc.store_scatter(oh, [rows16, e16], ones16)
            idx_list.append((rows16, e16))
        pltpu.sync_copy(oh, cnt_sh.at[dstb.at[0]], add=True)
        for rows16, e16 in idx_list:
            plsc.store_scatter(oh, [rows16, e16], zeros16)
        return 0

    lax.fori_loop(0, CPT, chunk, 0)
    plsc.subcore_barrier()
    pltpu.sync_copy(cnt_sh.at[pl.ds(s * NPT, NPT)],
                    cnt_out.at[c, pl.ds(s * NPT, NPT)])


_cnt_kernel = pl.kernel(
    _cnt_body,
    out_type=jax.ShapeDtypeStruct((NC, N, 16), jnp.float32),
    mesh=_mesh,
    compiler_params=pltpu.CompilerParams(use_tc_tiling_on_sc=False, needs_layout_passes=False),
    scratch_types=[
        pltpu.VMEM_SHARED((N, 16), jnp.float32),
        pltpu.VMEM((1, CH), jnp.int32),
        pltpu.VMEM((1, CH), jnp.int32),
        pltpu.VMEM((CH, 16), jnp.float32),
    ],
)


def _prep_tc_body(cnt_ref, batch_ref, inv_ref, offs_ref, cntg_ref):
    cnt = cnt_ref[0, :, :R] + cnt_ref[1, :, :R]
    inv_ref[...] = 1.0 / jnp.maximum(cnt, 1.0)
    b = batch_ref[...]
    gid = lax.broadcasted_iota(jnp.int32, (1, G), 1)
    onehot = (b[:, None] == gid).astype(jnp.float32)
    counts = jnp.sum(onehot, axis=0)
    tri = (lax.broadcasted_iota(jnp.int32, (G, G), 0)
           <= lax.broadcasted_iota(jnp.int32, (G, G), 1)).astype(jnp.float32)
    csum = jnp.dot(counts, tri, preferred_element_type=jnp.float32, precision=lax.Precision.HIGHEST)
    offs = jnp.concatenate(
        [jnp.zeros((1,), jnp.float32), csum,
         jnp.full((15,), float(N), jnp.float32)]).astype(jnp.int32)
    offs_ref[...] = offs
    invg = jnp.where(counts > 0, 1.0 / jnp.maximum(counts, 1.0), 0.0)
    cntg_ref[...] = jnp.concatenate([invg, jnp.zeros((16,), jnp.float32)])


_prep_tc = pl.pallas_call(
    _prep_tc_body,
    out_shape=(
        jax.ShapeDtypeStruct((N, R), jnp.float32),
        jax.ShapeDtypeStruct((G + 16, ), jnp.int32),
        jax.ShapeDtypeStruct((G + 16,), jnp.float32),
    ),
)


def _prep_w_body(src2d, dst2d, et2d, inv_hbm, gidx_out, w_out,
                 invbuf, srcb, dstb, etb, gidxb, wb):
    wid = _wid()
    pltpu.sync_copy(inv_hbm, invbuf)

    def chunk(j, _):
        row = wid * CPT + j
        pltpu.sync_copy(src2d.at[row], srcb.at[0])
        pltpu.sync_copy(dst2d.at[row], dstb.at[0])
        pltpu.sync_copy(et2d.at[row], etb.at[0])
        for gi in range(CH // 16):
            sl = pl.ds(gi * 16, 16)
            s16 = srcb[0, sl]
            d16 = dstb[0, sl]
            e16 = etb[0, sl]
            mask = e16 < R
            gidxb[0, sl] = jnp.where(mask, e16 * N + s16, 0)
            pidx = d16 * R + e16
            w16 = plsc.load_gather(invbuf, [pidx])
            wb[sl] = jnp.where(mask, w16, 0.0)
        pltpu.sync_copy(gidxb.at[0], gidx_out.at[row])
        pltpu.sync_copy(wb, w_out.at[row])
        return 0

    lax.fori_loop(0, CPT, chunk, 0)


_prep_w = pl.kernel(
    _prep_w_body,
    out_type=(
        jax.ShapeDtypeStruct((NROWS, CH), jnp.int32),
        jax.ShapeDtypeStruct((NROWS, CH), jnp.float32),
    ),
    mesh=_mesh,
    compiler_params=pltpu.CompilerParams(use_tc_tiling_on_sc=False, needs_layout_passes=False),
    scratch_types=[
        pltpu.VMEM((N * R,), jnp.float32),
        pltpu.VMEM((1, CH), jnp.int32),
        pltpu.VMEM((1, CH), jnp.int32),
        pltpu.VMEM((1, CH), jnp.int32),
        pltpu.VMEM((1, CH), jnp.int32),
        pltpu.VMEM((CH,), jnp.float32),
    ],
)


def _msg_body(h_hbm, gidx2d, w2d, dst2d, za, agg_out,
              acc_sh, rows_a, rows_b, gidxb, dstb, wb, semg, sems):
    c = lax.axis_index("c")
    s = lax.axis_index("s")
    wid = _wid()
    pltpu.sync_copy(za.at[pl.ds(s * NPT, NPT)], acc_sh.at[pl.ds(s * NPT, NPT)])
    plsc.subcore_barrier()

    base = wid * CPT
    pltpu.sync_copy(gidx2d.at[pl.ds(base, CPT)], gidxb)
    pltpu.sync_copy(dst2d.at[pl.ds(base, CPT)], dstb)
    pltpu.sync_copy(w2d.at[pl.ds(base, CPT)], wb)

    def scale(buf, j):
        def body(i, _):
            wsp = plsc.load_gather(
                wb, [jnp.full((16,), j, jnp.int32), jnp.full((16,), i, jnp.int32)])
            for q in range(H // 16):
                sl = pl.ds(q * 16, 16)
                buf[i, sl] = buf[i, sl] * wsp
            return 0
        lax.fori_loop(0, CH, body, 0, unroll=4)

    pltpu.async_copy(h_hbm.at[gidxb.at[0]], rows_a, semg)

    def outer(j2, _):
        for b in range(2):
            cur, oth = (rows_a, rows_b) if b == 0 else (rows_b, rows_a)
            j = j2 + b
            pltpu.make_async_copy(h_hbm.at[gidxb.at[0]], cur, semg).wait()

            @pl.when(j >= 1)
            def _():
                pltpu.make_async_copy(oth, acc_sh.at[dstb.at[0]], sems).wait()

            @pl.when(j + 1 < CPT)
            def _():
                pltpu.async_copy(h_hbm.at[gidxb.at[j + 1]], oth, semg)

            scale(cur, j)
            pltpu.async_copy(cur, acc_sh.at[dstb.at[j]], sems, add=True)
        return 0

    lax.fori_loop(0, CPT // 2, lambda t, u: outer(t * 2, u), 0)
    pltpu.make_async_copy(rows_b, acc_sh.at[dstb.at[0]], sems).wait()
    plsc.subcore_barrier()
    pltpu.sync_copy(acc_sh.at[pl.ds(s * NPT, NPT)],
                    agg_out.at[c, pl.ds(s * NPT, NPT)])


_msg_kernel = pl.kernel(
    _msg_body,
    out_type=jax.ShapeDtypeStruct((NC, N, H), jnp.float32),
    mesh=_mesh,
    compiler_params=pltpu.CompilerParams(use_tc_tiling_on_sc=False, needs_layout_passes=False),
    scratch_types=[
        pltpu.VMEM_SHARED((N, H), jnp.float32),
        pltpu.VMEM((CH, H), jnp.float32),
        pltpu.VMEM((CH, H), jnp.float32),
        pltpu.VMEM((CPT, CH), jnp.int32),
        pltpu.VMEM((CPT, CH), jnp.int32),
        pltpu.VMEM((CPT, CH), jnp.float32),
        pltpu.SemaphoreType.DMA,
        pltpu.SemaphoreType.DMA,
    ],
)


RT = 2000
NT = N // RT


def _h_body(x_ref, wrel_ref, wroot_ref, b_ref, h_ref, root_ref):
    r = pl.program_id(1)
    h_ref[0] = jnp.dot(x_ref[...], wrel_ref[0],
                       preferred_element_type=jnp.float32)

    @pl.when(r == 0)
    def _():
        root_ref[...] = (jnp.dot(x_ref[...], wroot_ref[...],
                                 preferred_element_type=jnp.float32)
                         + b_ref[...][None, :])


def _make_h_kernel(din):
    return pl.pallas_call(
        _h_body,
        grid=(NT, R),
        in_specs=[
            pl.BlockSpec((RT, din), lambda t, r: (t, 0)),
            pl.BlockSpec((1, din, H), lambda t, r: (r, 0, 0)),
            pl.BlockSpec((din, H), lambda t, r: (0, 0)),
            pl.BlockSpec((H,), lambda t, r: (0,)),
        ],
        out_specs=[
            pl.BlockSpec((1, RT, H), lambda t, r: (r, t, 0)),
            pl.BlockSpec((RT, H), lambda t, r: (t, 0)),
        ],
        out_shape=(
            jax.ShapeDtypeStruct((R, N, H), jnp.float32),
            jax.ShapeDtypeStruct((N, H), jnp.float32),
        ),
    )


_h_kernel_1 = _make_h_kernel(D)
_h_kernel_2 = _make_h_kernel(H)


def _bn_body(agg_ref, root_ref, g_ref, be_ref, out_ref):
    a = agg_ref[0] + agg_ref[1] + root_ref[...]
    m = jnp.mean(a, axis=0)
    v = jnp.mean((a - m[None, :]) ** 2, axis=0)
    y = (a - m[None, :]) / jnp.sqrt(v[None, :] + EPS) * g_ref[...][None, :] + be_ref[...][None, :]
    y = jnp.maximum(y, 0.0)
    out_ref[...] = jnp.concatenate(
        [y, jnp.zeros((NPAD - N, H), jnp.float32)], axis=0)


_bn_kernel = pl.pallas_call(
    _bn_body,
    out_shape=jax.ShapeDtypeStruct((NPAD, H), jnp.float32),
)


def _pool_body(x1, x2, x3, offs_hbm, cntg_hbm, out_hbm,
               offb, cntb, b1, b2, b3, outb, sem):
    wid = _wid()
    pltpu.sync_copy(offs_hbm, offb.at[0])
    pltpu.sync_copy(cntg_hbm, cntb)
    NQ = (3 * H) // 16

    for k in range(G // NW):
        g = wid * (G // NW) + k
        ovec = offb[0, pl.ds(g, 16)]
        s0 = ovec[0]
        e0 = ovec[1]
        nch = (e0 - s0 + 63) // 64

        def chunk(ci, carry):
            sums, maxs = carry
            base = s0 + ci * 64
            cp1 = pltpu.async_copy(x1.at[pl.ds(base, 64)], b1, sem)
            cp2 = pltpu.async_copy(x2.at[pl.ds(base, 64)], b2, sem)
            cp3 = pltpu.async_copy(x3.at[pl.ds(base, 64)], b3, sem)
            cp1.wait(); cp2.wait(); cp3.wait()

            def row(i, carry2):
                sums2, maxs2 = carry2
                vals = []
                for q in range(4):
                    vals.append(b1[i, pl.ds(q * 16, 16)])
                for q in range(4):
                    vals.append(b2[i, pl.ds(q * 16, 16)])
                for q in range(4):
                    vals.append(b3[i, pl.ds(q * 16, 16)])
                act = (base + i) < e0
                ns = tuple(jnp.where(act, s + v, s) for s, v in zip(sums2, vals))
                nm = tuple(jnp.where(act, jnp.maximum(m, v), m)
                           for m, v in zip(maxs2, vals))
                return (ns, nm)

            return lax.fori_loop(0, 64, row, (sums, maxs))

        zero16 = jnp.zeros((16,), jnp.float32)
        ninf16 = jnp.full((16,), -jnp.inf, jnp.float32)
        sums0 = tuple(zero16 for _ in range(NQ))
        maxs0 = tuple(ninf16 for _ in range(NQ))
        sums, maxs = lax.fori_loop(0, nch, chunk, (sums0, maxs0))

        scale = cntb[pl.ds(g, 16)][0]
        nonempty = scale > 0.0
        for q in range(NQ):
            outb[0, pl.ds(q * 16, 16)] = sums[q] * scale
            outb[0, pl.ds((NQ + q) * 16, 16)] = jnp.where(
                nonempty, maxs[q], jnp.zeros((16,), jnp.float32))
        pltpu.sync_copy(outb.at[0], out_hbm.at[g])


_pool_kernel = pl.kernel(
    _pool_body,
    out_type=jax.ShapeDtypeStruct((G, 6 * H), jnp.float32),
    mesh=_mesh,
    compiler_params=pltpu.CompilerParams(use_tc_tiling_on_sc=False, needs_layout_passes=False),
    scratch_types=[
        pltpu.VMEM((1, G + 16), jnp.int32),
        pltpu.VMEM((G + 16,), jnp.float32),
        pltpu.VMEM((64, H), jnp.float32),
        pltpu.VMEM((64, H), jnp.float32),
        pltpu.VMEM((64, H), jnp.float32),
        pltpu.VMEM((1, 6 * H), jnp.float32),
        pltpu.SemaphoreType.DMA,
    ],
)


def _head_body(pool_ref, wf1_ref, bf1_ref, gf_ref, bef_ref, wf2_ref, bf2_ref,
               wd1_ref, bd1_ref, wd2_ref, bd2_ref, pred_ref, emb_ref):
    xg = pool_ref[...]
    h = jnp.dot(xg, wf1_ref[...], preferred_element_type=jnp.float32) + bf1_ref[...][None, :]
    m = jnp.mean(h, axis=0)
    v = jnp.mean((h - m[None, :]) ** 2, axis=0)
    h = (h - m[None, :]) / jnp.sqrt(v[None, :] + EPS) * gf_ref[...][None, :] + bef_ref[...][None, :]
    h = jnp.where(h > 0, h, 0.2 * h)
    emb = jnp.dot(h, wf2_ref[...], preferred_element_type=jnp.float32) + bf2_ref[...][None, :]
    t = jnp.dot(emb, wd1_ref[...], preferred_element_type=jnp.float32) + bd1_ref[...][None, :]
    t = jnp.where(t > 0, t, 0.2 * t)
    pred_ref[...] = jnp.dot(t, wd2_ref[...], preferred_element_type=jnp.float32) + bd2_ref[...][None, :]
    emb_ref[...] = emb


_head_kernel = pl.pallas_call(
    _head_body,
    out_shape=(
        jax.ShapeDtypeStruct((G, OUT), jnp.float32),
        jax.ShapeDtypeStruct((G, EMB), jnp.float32),
    ),
)


def kernel(x, edge_index, edge_type, batch,
           W1_rel, W1_root, b1, g1, be1,
           W2_rel, W2_root, b2, g2, be2,
           W3_rel, W3_root, b3, g3, be3,
           Wf1, bf1, gf, bef, Wf2, bf2, Wd1, bd1, Wd2, bd2):
    src = edge_index[0]
    dst = edge_index[1]
    pad = E_PAD - E
    src2d = jnp.pad(src, (0, pad)).reshape(NROWS, CH)
    dst2d = jnp.pad(dst, (0, pad)).reshape(NROWS, CH)
    et2d = jnp.pad(edge_type, (0, pad), constant_values=15).reshape(NROWS, CH)
    zc = jnp.zeros((N, 16), jnp.float32)
    za = jnp.zeros((N, H), jnp.float32)

    cnt_part = _cnt_kernel(dst2d, et2d, zc)
    inv2, offs, cntg = _prep_tc(cnt_part, batch)
    gidx2d, w2d = _prep_w(src2d, dst2d, et2d, inv2.reshape(-1))

    def layer(h_kernel, xprev, W_rel, W_root, b, g, be):
        h, root = h_kernel(xprev[:N], W_rel, W_root, b)
        agg_part = _msg_kernel(h.reshape(R * N, H), gidx2d, w2d, dst2d, za)
        return _bn_kernel(agg_part, root, g, be)

    x1 = layer(_h_kernel_1, x, W1_rel, W1_root, b1, g1, be1)
    x2 = layer(_h_kernel_2, x1, W2_rel, W2_root, b2, g2, be2)
    x3 = layer(_h_kernel_2, x2, W3_rel, W3_root, b3, g3, be3)

    pooled = _pool_kernel(x1, x2, x3, offs, cntg)
    pred, emb = _head_kernel(pooled, Wf1, bf1, gf, bef, Wf2, bf2,
                             Wd1, bd1, Wd2, bd2)
    return (pred, emb)


# speedup vs baseline: 16.3945x; 1.0590x over previous
import functools

import jax
import jax.numpy as jnp
from jax import lax
from jax.experimental import pallas as pl
from jax.experimental.pallas import tpu as pltpu
from jax.experimental.pallas import tpu_sc as plsc

N = 10000
E = 320000
D = 128
R = 8
H = 64
G = 128
EMB = 16
OUT = 8
EPS = 1e-5

NC = 2
NS = 16
NW = NC * NS
CH = 128
CPT = 80
E_PAD = NW * CPT * CH
NROWS = E_PAD // CH
NPT = N // NS
NPAD = N + 64

_mesh = plsc.VectorSubcoreMesh(
    core_axis_name="c", subcore_axis_name="s", num_cores=NC, num_subcores=NS)


def _wid():
    return lax.axis_index("s") * NC + lax.axis_index("c")


def _cnt_body(dst2d, et2d, zc, cnt_out, cnt_sh, dstb, etb, oh):
    c = lax.axis_index("c")
    s = lax.axis_index("s")
    wid = _wid()
    pltpu.sync_copy(zc.at[pl.ds(s * NPT, NPT)], cnt_sh.at[pl.ds(s * NPT, NPT)])
    plsc.subcore_barrier()

    ones16 = jnp.full((16,), 1.0, jnp.float32)
    zeros16 = jnp.zeros((16,), jnp.float32)

    def zrow(i, _):
        oh[i, :] = zeros16
        return 0

    lax.fori_loop(0, CH, zrow, 0)

    def chunk(j, _):
        row = wid * CPT + j
        pltpu.sync_copy(dst2d.at[row], dstb.at[0])
        pltpu.sync_copy(et2d.at[row], etb.at[0])
        idx_list = []
        for gi in range(CH // 16):
            rows16 = lax.iota(jnp.int32, 16) + gi * 16
            e16 = etb[0, pl.ds(gi * 16, 16)]
            plsc.store_scatter(oh, [rows16, e16], ones16)
            idx_list.append((rows16, e16))
        pltpu.sync_copy(oh, cnt_sh.at[dstb.at[0]], add=True)
        for rows16, e16 in idx_list:
            plsc.store_scatter(oh, [rows16, e16], zeros16)
        return 0

    lax.fori_loop(0, CPT, chunk, 0)
    plsc.subcore_barrier()
    pltpu.sync_copy(cnt_sh.at[pl.ds(s * NPT, NPT)],
                    cnt_out.at[c, pl.ds(s * NPT, NPT)])


_cnt_kernel = pl.kernel(
    _cnt_body,
    out_type=jax.ShapeDtypeStruct((NC, N, 16), jnp.float32),
    mesh=_mesh,
    compiler_params=pltpu.CompilerParams(use_tc_tiling_on_sc=False, needs_layout_passes=False),
    scratch_types=[
        pltpu.VMEM_SHARED((N, 16), jnp.float32),
        pltpu.VMEM((1, CH), jnp.int32),
        pltpu.VMEM((1, CH), jnp.int32),
        pltpu.VMEM((CH, 16), jnp.float32),
    ],
)


def _prep_tc_body(cnt_ref, batch_ref, inv_ref, offs_ref, cntg_ref):
    cnt = cnt_ref[0, :, :R] + cnt_ref[1, :, :R]
    inv_ref[...] = 1.0 / jnp.maximum(cnt, 1.0)
    b = batch_ref[...]
    gid = lax.broadcasted_iota(jnp.int32, (1, G), 1)
    onehot = (b[:, None] == gid).astype(jnp.float32)
    counts = jnp.sum(onehot, axis=0)
    tri = (lax.broadcasted_iota(jnp.int32, (G, G), 0)
           <= lax.broadcasted_iota(jnp.int32, (G, G), 1)).astype(jnp.float32)
    csum = jnp.dot(counts, tri, preferred_element_type=jnp.float32, precision=lax.Precision.HIGHEST)
    offs = jnp.concatenate(
        [jnp.zeros((1,), jnp.float32), csum,
         jnp.full((15,), float(N), jnp.float32)]).astype(jnp.int32)
    offs_ref[...] = offs
    invg = jnp.where(counts > 0, 1.0 / jnp.maximum(counts, 1.0), 0.0)
    cntg_ref[...] = jnp.concatenate([invg, jnp.zeros((16,), jnp.float32)])


_prep_tc = pl.pallas_call(
    _prep_tc_body,
    out_shape=(
        jax.ShapeDtypeStruct((N, R), jnp.float32),
        jax.ShapeDtypeStruct((G + 16, ), jnp.int32),
        jax.ShapeDtypeStruct((G + 16,), jnp.float32),
    ),
)


def _prep_w_body(src2d, dst2d, et2d, inv_hbm, gidx_out, w_out,
                 invbuf, srcb, dstb, etb, gidxb, wb):
    wid = _wid()
    pltpu.sync_copy(inv_hbm, invbuf)

    def chunk(j, _):
        row = wid * CPT + j
        pltpu.sync_copy(src2d.at[row], srcb.at[0])
        pltpu.sync_copy(dst2d.at[row], dstb.at[0])
        pltpu.sync_copy(et2d.at[row], etb.at[0])
        for gi in range(CH // 16):
            sl = pl.ds(gi * 16, 16)
            s16 = srcb[0, sl]
            d16 = dstb[0, sl]
            e16 = etb[0, sl]
            mask = e16 < R
            gidxb[0, sl] = jnp.where(mask, e16 * N + s16, 0)
            pidx = d16 * R + e16
            w16 = plsc.load_gather(invbuf, [pidx])
            wb[sl] = jnp.where(mask, w16, 0.0)
        pltpu.sync_copy(gidxb.at[0], gidx_out.at[row])
        pltpu.sync_copy(wb, w_out.at[row])
        return 0

    lax.fori_loop(0, CPT, chunk, 0)


_prep_w = pl.kernel(
    _prep_w_body,
    out_type=(
        jax.ShapeDtypeStruct((NROWS, CH), jnp.int32),
        jax.ShapeDtypeStruct((NROWS, CH), jnp.float32),
    ),
    mesh=_mesh,
    compiler_params=pltpu.CompilerParams(use_tc_tiling_on_sc=False, needs_layout_passes=False),
    scratch_types=[
        pltpu.VMEM((N * R,), jnp.float32),
        pltpu.VMEM((1, CH), jnp.int32),
        pltpu.VMEM((1, CH), jnp.int32),
        pltpu.VMEM((1, CH), jnp.int32),
        pltpu.VMEM((1, CH), jnp.int32),
        pltpu.VMEM((CH,), jnp.float32),
    ],
)


def _msg_body(h_hbm, gidx2d, w2d, dst2d, za, agg_out,
              acc_sh, rows_a, rows_b, rows_c, rows_d, gidxb, dstb, wb,
              semg, sems):
    c = lax.axis_index("c")
    s = lax.axis_index("s")
    wid = _wid()
    pltpu.sync_copy(za.at[pl.ds(s * NPT, NPT)], acc_sh.at[pl.ds(s * NPT, NPT)])
    plsc.subcore_barrier()

    base = wid * CPT
    pltpu.sync_copy(gidx2d.at[pl.ds(base, CPT)], gidxb)
    pltpu.sync_copy(dst2d.at[pl.ds(base, CPT)], dstb)
    pltpu.sync_copy(w2d.at[pl.ds(base, CPT)], wb)

    def scale(buf, j):
        def body(i, _):
            wsp = plsc.load_gather(
                wb, [jnp.full((16,), j, jnp.int32), jnp.full((16,), i, jnp.int32)])
            for q in range(H // 16):
                sl = pl.ds(q * 16, 16)
                buf[i, sl] = buf[i, sl] * wsp
            return 0
        lax.fori_loop(0, CH, body, 0, unroll=4)

    bufs = (rows_a, rows_b, rows_c, rows_d)
    for p in range(3):
        pltpu.async_copy(h_hbm.at[gidxb.at[p]], bufs[p], semg)

    def outer(j4, _):
        for b in range(4):
            cur = bufs[b]
            nxt = bufs[(b + 3) % 4]
            j = j4 + b
            pltpu.make_async_copy(h_hbm.at[gidxb.at[0]], cur, semg).wait()

            @pl.when(j + 3 < CPT)
            def _():
                @pl.when(j >= 1)
                def _():
                    pltpu.make_async_copy(nxt, acc_sh.at[dstb.at[0]], sems).wait()
                pltpu.async_copy(h_hbm.at[gidxb.at[j + 3]], nxt, semg)

            scale(cur, j)
            pltpu.async_copy(cur, acc_sh.at[dstb.at[j]], sems, add=True)
        return 0

    lax.fori_loop(0, CPT // 4, lambda t, u: outer(t * 4, u), 0)
    for p in range(4):
        pltpu.make_async_copy(bufs[p], acc_sh.at[dstb.at[0]], sems).wait()
    plsc.subcore_barrier()
    pltpu.sync_copy(acc_sh.at[pl.ds(s * NPT, NPT)],
                    agg_out.at[c, pl.ds(s * NPT, NPT)])


_msg_kernel = pl.kernel(
    _msg_body,
    out_type=jax.ShapeDtypeStruct((NC, N, H), jnp.float32),
    mesh=_mesh,
    compiler_params=pltpu.CompilerParams(use_tc_tiling_on_sc=False, needs_layout_passes=False),
    scratch_types=[
        pltpu.VMEM_SHARED((N, H), jnp.float32),
        pltpu.VMEM((CH, H), jnp.float32),
        pltpu.VMEM((CH, H), jnp.float32),
        pltpu.VMEM((CH, H), jnp.float32),
        pltpu.VMEM((CH, H), jnp.float32),
        pltpu.VMEM((CPT, CH), jnp.int32),
        pltpu.VMEM((CPT, CH), jnp.int32),
        pltpu.VMEM((CPT, CH), jnp.float32),
        pltpu.SemaphoreType.DMA,
        pltpu.SemaphoreType.DMA,
    ],
)


RT = 2000
NT = N // RT


def _h_body(x_ref, wrel_ref, wroot_ref, b_ref, h_ref, root_ref):
    r = pl.program_id(1)
    h_ref[0] = jnp.dot(x_ref[...], wrel_ref[0],
                       preferred_element_type=jnp.float32)

    @pl.when(r == 0)
    def _():
        root_ref[...] = (jnp.dot(x_ref[...], wroot_ref[...],
                                 preferred_element_type=jnp.float32)
                         + b_ref[...][None, :])


def _make_h_kernel(din):
    return pl.pallas_call(
        _h_body,
        grid=(NT, R),
        in_specs=[
            pl.BlockSpec((RT, din), lambda t, r: (t, 0)),
            pl.BlockSpec((1, din, H), lambda t, r: (r, 0, 0)),
            pl.BlockSpec((din, H), lambda t, r: (0, 0)),
            pl.BlockSpec((H,), lambda t, r: (0,)),
        ],
        out_specs=[
            pl.BlockSpec((1, RT, H), lambda t, r: (r, t, 0)),
            pl.BlockSpec((RT, H), lambda t, r: (t, 0)),
        ],
        out_shape=(
            jax.ShapeDtypeStruct((R, N, H), jnp.float32),
            jax.ShapeDtypeStruct((N, H), jnp.float32),
        ),
    )


_h_kernel_1 = _make_h_kernel(D)
_h_kernel_2 = _make_h_kernel(H)


def _bn_body(agg_ref, root_ref, g_ref, be_ref, out_ref):
    a = agg_ref[0] + agg_ref[1] + root_ref[...]
    m = jnp.mean(a, axis=0)
    v = jnp.mean((a - m[None, :]) ** 2, axis=0)
    y = (a - m[None, :]) / jnp.sqrt(v[None, :] + EPS) * g_ref[...][None, :] + be_ref[...][None, :]
    y = jnp.maximum(y, 0.0)
    out_ref[...] = jnp.concatenate(
        [y, jnp.zeros((NPAD - N, H), jnp.float32)], axis=0)


_bn_kernel = pl.pallas_call(
    _bn_body,
    out_shape=jax.ShapeDtypeStruct((NPAD, H), jnp.float32),
)


def _pool_body(x1, x2, x3, offs_hbm, cntg_hbm, out_hbm,
               offb, cntb, b1, b2, b3, outb, sem):
    wid = _wid()
    pltpu.sync_copy(offs_hbm, offb.at[0])
    pltpu.sync_copy(cntg_hbm, cntb)
    NQ = (3 * H) // 16

    for k in range(G // NW):
        g = wid * (G // NW) + k
        ovec = offb[0, pl.ds(g, 16)]
        s0 = ovec[0]
        e0 = ovec[1]
        nch = (e0 - s0 + 63) // 64

        def chunk(ci, carry):
            sums, maxs = carry
            base = s0 + ci * 64
            cp1 = pltpu.async_copy(x1.at[pl.ds(base, 64)], b1, sem)
            cp2 = pltpu.async_copy(x2.at[pl.ds(base, 64)], b2, sem)
            cp3 = pltpu.async_copy(x3.at[pl.ds(base, 64)], b3, sem)
            cp1.wait(); cp2.wait(); cp3.wait()

            def row(i, carry2):
                sums2, maxs2 = carry2
                vals = []
                for q in range(4):
                    vals.append(b1[i, pl.ds(q * 16, 16)])
                for q in range(4):
                    vals.append(b2[i, pl.ds(q * 16, 16)])
                for q in range(4):
                    vals.append(b3[i, pl.ds(q * 16, 16)])
                act = (base + i) < e0
                ns = tuple(jnp.where(act, s + v, s) for s, v in zip(sums2, vals))
                nm = tuple(jnp.where(act, jnp.maximum(m, v), m)
                           for m, v in zip(maxs2, vals))
                return (ns, nm)

            return lax.fori_loop(0, 64, row, (sums, maxs))

        zero16 = jnp.zeros((16,), jnp.float32)
        ninf16 = jnp.full((16,), -jnp.inf, jnp.float32)
        sums0 = tuple(zero16 for _ in range(NQ))
        maxs0 = tuple(ninf16 for _ in range(NQ))
        sums, maxs = lax.fori_loop(0, nch, chunk, (sums0, maxs0))

        scale = cntb[pl.ds(g, 16)][0]
        nonempty = scale > 0.0
        for q in range(NQ):
            outb[0, pl.ds(q * 16, 16)] = sums[q] * scale
            outb[0, pl.ds((NQ + q) * 16, 16)] = jnp.where(
                nonempty, maxs[q], jnp.zeros((16,), jnp.float32))
        pltpu.sync_copy(outb.at[0], out_hbm.at[g])


_pool_kernel = pl.kernel(
    _pool_body,
    out_type=jax.ShapeDtypeStruct((G, 6 * H), jnp.float32),
    mesh=_mesh,
    compiler_params=pltpu.CompilerParams(use_tc_tiling_on_sc=False, needs_layout_passes=False),
    scratch_types=[
        pltpu.VMEM((1, G + 16), jnp.int32),
        pltpu.VMEM((G + 16,), jnp.float32),
        pltpu.VMEM((64, H), jnp.float32),
        pltpu.VMEM((64, H), jnp.float32),
        pltpu.VMEM((64, H), jnp.float32),
        pltpu.VMEM((1, 6 * H), jnp.float32),
        pltpu.SemaphoreType.DMA,
    ],
)


def _head_body(pool_ref, wf1_ref, bf1_ref, gf_ref, bef_ref, wf2_ref, bf2_ref,
               wd1_ref, bd1_ref, wd2_ref, bd2_ref, pred_ref, emb_ref):
    xg = pool_ref[...]
    h = jnp.dot(xg, wf1_ref[...], preferred_element_type=jnp.float32) + bf1_ref[...][None, :]
    m = jnp.mean(h, axis=0)
    v = jnp.mean((h - m[None, :]) ** 2, axis=0)
    h = (h - m[None, :]) / jnp.sqrt(v[None, :] + EPS) * gf_ref[...][None, :] + bef_ref[...][None, :]
    h = jnp.where(h > 0, h, 0.2 * h)
    emb = jnp.dot(h, wf2_ref[...], preferred_element_type=jnp.float32) + bf2_ref[...][None, :]
    t = jnp.dot(emb, wd1_ref[...], preferred_element_type=jnp.float32) + bd1_ref[...][None, :]
    t = jnp.where(t > 0, t, 0.2 * t)
    pred_ref[...] = jnp.dot(t, wd2_ref[...], preferred_element_type=jnp.float32) + bd2_ref[...][None, :]
    emb_ref[...] = emb


_head_kernel = pl.pallas_call(
    _head_body,
    out_shape=(
        jax.ShapeDtypeStruct((G, OUT), jnp.float32),
        jax.ShapeDtypeStruct((G, EMB), jnp.float32),
    ),
)


def kernel(x, edge_index, edge_type, batch,
           W1_rel, W1_root, b1, g1, be1,
           W2_rel, W2_root, b2, g2, be2,
           W3_rel, W3_root, b3, g3, be3,
           Wf1, bf1, gf, bef, Wf2, bf2, Wd1, bd1, Wd2, bd2):
    src = edge_index[0]
    dst = edge_index[1]
    pad = E_PAD - E
    src2d = jnp.pad(src, (0, pad)).reshape(NROWS, CH)
    dst2d = jnp.pad(dst, (0, pad)).reshape(NROWS, CH)
    et2d = jnp.pad(edge_type, (0, pad), constant_values=15).reshape(NROWS, CH)
    zc = jnp.zeros((N, 16), jnp.float32)
    za = jnp.zeros((N, H), jnp.float32)

    cnt_part = _cnt_kernel(dst2d, et2d, zc)
    inv2, offs, cntg = _prep_tc(cnt_part, batch)
    gidx2d, w2d = _prep_w(src2d, dst2d, et2d, inv2.reshape(-1))

    def layer(h_kernel, xprev, W_rel, W_root, b, g, be):
        h, root = h_kernel(xprev[:N], W_rel, W_root, b)
        agg_part = _msg_kernel(h.reshape(R * N, H), gidx2d, w2d, dst2d, za)
        return _bn_kernel(agg_part, root, g, be)

    x1 = layer(_h_kernel_1, x, W1_rel, W1_root, b1, g1, be1)
    x2 = layer(_h_kernel_2, x1, W2_rel, W2_root, b2, g2, be2)
    x3 = layer(_h_kernel_2, x2, W3_rel, W3_root, b3, g3, be3)

    pooled = _pool_kernel(x1, x2, x3, offs, cntg)
    pred, emb = _head_kernel(pooled, Wf1, bf1, gf, bef, Wf2, bf2,
                             Wd1, bd1, Wd2, bd2)
    return (pred, emb)


# speedup vs baseline: 17.2908x; 1.0547x over previous
import functools

import jax
import jax.numpy as jnp
from jax import lax
from jax.experimental import pallas as pl
from jax.experimental.pallas import tpu as pltpu
from jax.experimental.pallas import tpu_sc as plsc

N = 10000
E = 320000
D = 128
R = 8
H = 64
G = 128
EMB = 16
OUT = 8
EPS = 1e-5

NC = 2
NS = 16
NW = NC * NS
CH = 128
CPT = 80
E_PAD = NW * CPT * CH
NROWS = E_PAD // CH
NPT = N // NS
NPAD = N + 64

_mesh = plsc.VectorSubcoreMesh(
    core_axis_name="c", subcore_axis_name="s", num_cores=NC, num_subcores=NS)


def _wid():
    return lax.axis_index("s") * NC + lax.axis_index("c")


def _cnt_body(dst2d, et2d, zc, cnt_out, cnt_sh, dstb, etb, oh):
    c = lax.axis_index("c")
    s = lax.axis_index("s")
    wid = _wid()
    pltpu.sync_copy(zc.at[pl.ds(s * NPT, NPT)], cnt_sh.at[pl.ds(s * NPT, NPT)])
    plsc.subcore_barrier()

    ones16 = jnp.full((16,), 1.0, jnp.float32)
    zeros16 = jnp.zeros((16,), jnp.float32)

    def zrow(i, _):
        oh[i, :] = zeros16
        return 0

    lax.fori_loop(0, CH, zrow, 0)

    def chunk(j, _):
        row = wid * CPT + j
        pltpu.sync_copy(dst2d.at[row], dstb.at[0])
        pltpu.sync_copy(et2d.at[row], etb.at[0])
        idx_list = []
        for gi in range(CH // 16):
            rows16 = lax.iota(jnp.int32, 16) + gi * 16
            e16 = etb[0, pl.ds(gi * 16, 16)]
            plsc.store_scatter(oh, [rows16, e16], ones16)
            idx_list.append((rows16, e16))
        pltpu.sync_copy(oh, cnt_sh.at[dstb.at[0]], add=True)
        for rows16, e16 in idx_list:
            plsc.store_scatter(oh, [rows16, e16], zeros16)
        return 0

    lax.fori_loop(0, CPT, chunk, 0)
    plsc.subcore_barrier()
    pltpu.sync_copy(cnt_sh.at[pl.ds(s * NPT, NPT)],
                    cnt_out.at[c, pl.ds(s * NPT, NPT)])


_cnt_kernel = pl.kernel(
    _cnt_body,
    out_type=jax.ShapeDtypeStruct((NC, N, 16), jnp.float32),
    mesh=_mesh,
    compiler_params=pltpu.CompilerParams(use_tc_tiling_on_sc=False, needs_layout_passes=False),
    scratch_types=[
        pltpu.VMEM_SHARED((N, 16), jnp.float32),
        pltpu.VMEM((1, CH), jnp.int32),
        pltpu.VMEM((1, CH), jnp.int32),
        pltpu.VMEM((CH, 16), jnp.float32),
    ],
)


def _prep_tc_body(cnt_ref, batch_ref, inv_ref, offs_ref, cntg_ref):
    cnt = cnt_ref[0, :, :R] + cnt_ref[1, :, :R]
    inv_ref[...] = 1.0 / jnp.maximum(cnt, 1.0)
    b = batch_ref[...]
    gid = lax.broadcasted_iota(jnp.int32, (1, G), 1)
    onehot = (b[:, None] == gid).astype(jnp.float32)
    counts = jnp.sum(onehot, axis=0)
    tri = (lax.broadcasted_iota(jnp.int32, (G, G), 0)
           <= lax.broadcasted_iota(jnp.int32, (G, G), 1)).astype(jnp.float32)
    csum = jnp.dot(counts, tri, preferred_element_type=jnp.float32, precision=lax.Precision.HIGHEST)
    offs = jnp.concatenate(
        [jnp.zeros((1,), jnp.float32), csum,
         jnp.full((15,), float(N), jnp.float32)]).astype(jnp.int32)
    offs_ref[...] = offs
    invg = jnp.where(counts > 0, 1.0 / jnp.maximum(counts, 1.0), 0.0)
    cntg_ref[...] = jnp.concatenate([invg, jnp.zeros((16,), jnp.float32)])


_prep_tc = pl.pallas_call(
    _prep_tc_body,
    out_shape=(
        jax.ShapeDtypeStruct((N, R), jnp.float32),
        jax.ShapeDtypeStruct((G + 16, ), jnp.int32),
        jax.ShapeDtypeStruct((G + 16,), jnp.float32),
    ),
)


def _prep_w_body(dst2d, et2d, inv_hbm, w_out, invbuf, dstb, etb, wb):
    wid = _wid()
    base = wid * CPT
    pltpu.sync_copy(inv_hbm, invbuf)
    pltpu.sync_copy(dst2d.at[pl.ds(base, CPT)], dstb)
    pltpu.sync_copy(et2d.at[pl.ds(base, CPT)], etb)

    def chunk(j, _):
        def grp(gi, _u):
            sl = pl.ds(gi * 16, 16)
            d16 = dstb[j, sl]
            e16 = etb[j, sl]
            pidx = d16 * R + e16
            w16 = plsc.load_gather(invbuf, [pidx])
            wb[j, sl] = jnp.where(e16 < R, w16, 0.0)
            return 0
        lax.fori_loop(0, CH // 16, grp, 0, unroll=4)
        return 0

    lax.fori_loop(0, CPT, chunk, 0)
    pltpu.sync_copy(wb, w_out.at[pl.ds(base, CPT)])


_prep_w = pl.kernel(
    _prep_w_body,
    out_type=jax.ShapeDtypeStruct((NROWS, CH), jnp.float32),
    mesh=_mesh,
    compiler_params=pltpu.CompilerParams(use_tc_tiling_on_sc=False, needs_layout_passes=False),
    scratch_types=[
        pltpu.VMEM((N * R,), jnp.float32),
        pltpu.VMEM((CPT, CH), jnp.int32),
        pltpu.VMEM((CPT, CH), jnp.int32),
        pltpu.VMEM((CPT, CH), jnp.float32),
    ],
)


def _msg_body(h_hbm, src2d, et2d, w2d, dst2d, za, agg_out,
              acc_sh, rows_a, rows_b, rows_c, rows_d, gidxb, dstb, wb, etb,
              semg, sems):
    c = lax.axis_index("c")
    s = lax.axis_index("s")
    wid = _wid()
    pltpu.sync_copy(za.at[pl.ds(s * NPT, NPT)], acc_sh.at[pl.ds(s * NPT, NPT)])
    plsc.subcore_barrier()

    base = wid * CPT
    pltpu.sync_copy(src2d.at[pl.ds(base, CPT)], gidxb)
    pltpu.sync_copy(dst2d.at[pl.ds(base, CPT)], dstb)
    pltpu.sync_copy(w2d.at[pl.ds(base, CPT)], wb)
    pltpu.sync_copy(et2d.at[pl.ds(base, CPT)], etb)

    def gix(j, _):
        def grp(gi, _u):
            sl = pl.ds(gi * 16, 16)
            e16 = etb[j, sl]
            s16 = gidxb[j, sl]
            gidxb[j, sl] = jnp.where(e16 < R, e16 * N + s16, 0)
            return 0
        lax.fori_loop(0, CH // 16, grp, 0, unroll=4)
        return 0

    lax.fori_loop(0, CPT, gix, 0)

    def scale(buf, j):
        def body(i, _):
            wsp = plsc.load_gather(
                wb, [jnp.full((16,), j, jnp.int32), jnp.full((16,), i, jnp.int32)])
            for q in range(H // 16):
                sl = pl.ds(q * 16, 16)
                buf[i, sl] = buf[i, sl] * wsp
            return 0
        lax.fori_loop(0, CH, body, 0, unroll=4)

    bufs = (rows_a, rows_b, rows_c, rows_d)
    for p in range(3):
        pltpu.async_copy(h_hbm.at[gidxb.at[p]], bufs[p], semg)

    def outer(j4, _):
        for b in range(4):
            cur = bufs[b]
            nxt = bufs[(b + 3) % 4]
            j = j4 + b
            pltpu.make_async_copy(h_hbm.at[gidxb.at[0]], cur, semg).wait()

            @pl.when(j + 3 < CPT)
            def _():
                @pl.when(j >= 1)
                def _():
                    pltpu.make_async_copy(nxt, acc_sh.at[dstb.at[0]], sems).wait()
                pltpu.async_copy(h_hbm.at[gidxb.at[j + 3]], nxt, semg)

            scale(cur, j)
            pltpu.async_copy(cur, acc_sh.at[dstb.at[j]], sems, add=True)
        return 0

    lax.fori_loop(0, CPT // 4, lambda t, u: outer(t * 4, u), 0)
    for p in range(4):
        pltpu.make_async_copy(bufs[p], acc_sh.at[dstb.at[0]], sems).wait()
    plsc.subcore_barrier()
    pltpu.sync_copy(acc_sh.at[pl.ds(s * NPT, NPT)],
                    agg_out.at[c, pl.ds(s * NPT, NPT)])


_msg_kernel = pl.kernel(
    _msg_body,
    out_type=jax.ShapeDtypeStruct((NC, N, H), jnp.float32),
    mesh=_mesh,
    compiler_params=pltpu.CompilerParams(use_tc_tiling_on_sc=False, needs_layout_passes=False),
    scratch_types=[
        pltpu.VMEM_SHARED((N, H), jnp.float32),
        pltpu.VMEM((CH, H), jnp.float32),
        pltpu.VMEM((CH, H), jnp.float32),
        pltpu.VMEM((CH, H), jnp.float32),
        pltpu.VMEM((CH, H), jnp.float32),
        pltpu.VMEM((CPT, CH), jnp.int32),
        pltpu.VMEM((CPT, CH), jnp.int32),
        pltpu.VMEM((CPT, CH), jnp.float32),
        pltpu.VMEM((CPT, CH), jnp.int32),
        pltpu.SemaphoreType.DMA,
        pltpu.SemaphoreType.DMA,
    ],
)


RT = 2000
NT = N // RT


def _h_body(x_ref, wrel_ref, wroot_ref, b_ref, h_ref, root_ref):
    r = pl.program_id(1)
    h_ref[0] = jnp.dot(x_ref[...], wrel_ref[0],
                       preferred_element_type=jnp.float32)

    @pl.when(r == 0)
    def _():
        root_ref[...] = (jnp.dot(x_ref[...], wroot_ref[...],
                                 preferred_element_type=jnp.float32)
                         + b_ref[...][None, :])


def _make_h_kernel(din):
    return pl.pallas_call(
        _h_body,
        grid=(NT, R),
        in_specs=[
            pl.BlockSpec((RT, din), lambda t, r: (t, 0)),
            pl.BlockSpec((1, din, H), lambda t, r: (r, 0, 0)),
            pl.BlockSpec((din, H), lambda t, r: (0, 0)),
            pl.BlockSpec((H,), lambda t, r: (0,)),
        ],
        out_specs=[
            pl.BlockSpec((1, RT, H), lambda t, r: (r, t, 0)),
            pl.BlockSpec((RT, H), lambda t, r: (t, 0)),
        ],
        out_shape=(
            jax.ShapeDtypeStruct((R, N, H), jnp.float32),
            jax.ShapeDtypeStruct((N, H), jnp.float32),
        ),
    )


_h_kernel_1 = _make_h_kernel(D)
_h_kernel_2 = _make_h_kernel(H)


def _bn_body(agg_ref, root_ref, g_ref, be_ref, out_ref):
    a = agg_ref[0] + agg_ref[1] + root_ref[...]
    m = jnp.mean(a, axis=0)
    v = jnp.mean((a - m[None, :]) ** 2, axis=0)
    y = (a - m[None, :]) / jnp.sqrt(v[None, :] + EPS) * g_ref[...][None, :] + be_ref[...][None, :]
    y = jnp.maximum(y, 0.0)
    out_ref[...] = jnp.concatenate(
        [y, jnp.zeros((NPAD - N, H), jnp.float32)], axis=0)


_bn_kernel = pl.pallas_call(
    _bn_body,
    out_shape=jax.ShapeDtypeStruct((NPAD, H), jnp.float32),
)


def _pool_body(x1, x2, x3, offs_hbm, cntg_hbm, out_hbm,
               offb, cntb, b1, b2, b3, outb, sem):
    wid = _wid()
    pltpu.sync_copy(offs_hbm, offb.at[0])
    pltpu.sync_copy(cntg_hbm, cntb)
    NQ = (3 * H) // 16

    for k in range(G // NW):
        g = wid * (G // NW) + k
        ovec = offb[0, pl.ds(g, 16)]
        s0 = ovec[0]
        e0 = ovec[1]
        nch = (e0 - s0 + 63) // 64

        def chunk(ci, carry):
            sums, maxs = carry
            base = s0 + ci * 64
            cp1 = pltpu.async_copy(x1.at[pl.ds(base, 64)], b1, sem)
            cp2 = pltpu.async_copy(x2.at[pl.ds(base, 64)], b2, sem)
            cp3 = pltpu.async_copy(x3.at[pl.ds(base, 64)], b3, sem)
            cp1.wait(); cp2.wait(); cp3.wait()

            def row(i, carry2):
                sums2, maxs2 = carry2
                vals = []
                for q in range(4):
                    vals.append(b1[i, pl.ds(q * 16, 16)])
                for q in range(4):
                    vals.append(b2[i, pl.ds(q * 16, 16)])
                for q in range(4):
                    vals.append(b3[i, pl.ds(q * 16, 16)])
                act = (base + i) < e0
                ns = tuple(jnp.where(act, s + v, s) for s, v in zip(sums2, vals))
                nm = tuple(jnp.where(act, jnp.maximum(m, v), m)
                           for m, v in zip(maxs2, vals))
                return (ns, nm)

            return lax.fori_loop(0, 64, row, (sums, maxs))

        zero16 = jnp.zeros((16,), jnp.float32)
        ninf16 = jnp.full((16,), -jnp.inf, jnp.float32)
        sums0 = tuple(zero16 for _ in range(NQ))
        maxs0 = tuple(ninf16 for _ in range(NQ))
        sums, maxs = lax.fori_loop(0, nch, chunk, (sums0, maxs0))

        scale = cntb[pl.ds(g, 16)][0]
        nonempty = scale > 0.0
        for q in range(NQ):
            outb[0, pl.ds(q * 16, 16)] = sums[q] * scale
            outb[0, pl.ds((NQ + q) * 16, 16)] = jnp.where(
                nonempty, maxs[q], jnp.zeros((16,), jnp.float32))
        pltpu.sync_copy(outb.at[0], out_hbm.at[g])


_pool_kernel = pl.kernel(
    _pool_body,
    out_type=jax.ShapeDtypeStruct((G, 6 * H), jnp.float32),
    mesh=_mesh,
    compiler_params=pltpu.CompilerParams(use_tc_tiling_on_sc=False, needs_layout_passes=False),
    scratch_types=[
        pltpu.VMEM((1, G + 16), jnp.int32),
        pltpu.VMEM((G + 16,), jnp.float32),
        pltpu.VMEM((64, H), jnp.float32),
        pltpu.VMEM((64, H), jnp.float32),
        pltpu.VMEM((64, H), jnp.float32),
        pltpu.VMEM((1, 6 * H), jnp.float32),
        pltpu.SemaphoreType.DMA,
    ],
)


def _head_body(pool_ref, wf1_ref, bf1_ref, gf_ref, bef_ref, wf2_ref, bf2_ref,
               wd1_ref, bd1_ref, wd2_ref, bd2_ref, pred_ref, emb_ref):
    xg = pool_ref[...]
    h = jnp.dot(xg, wf1_ref[...], preferred_element_type=jnp.float32) + bf1_ref[...][None, :]
    m = jnp.mean(h, axis=0)
    v = jnp.mean((h - m[None, :]) ** 2, axis=0)
    h = (h - m[None, :]) / jnp.sqrt(v[None, :] + EPS) * gf_ref[...][None, :] + bef_ref[...][None, :]
    h = jnp.where(h > 0, h, 0.2 * h)
    emb = jnp.dot(h, wf2_ref[...], preferred_element_type=jnp.float32) + bf2_ref[...][None, :]
    t = jnp.dot(emb, wd1_ref[...], preferred_element_type=jnp.float32) + bd1_ref[...][None, :]
    t = jnp.where(t > 0, t, 0.2 * t)
    pred_ref[...] = jnp.dot(t, wd2_ref[...], preferred_element_type=jnp.float32) + bd2_ref[...][None, :]
    emb_ref[...] = emb


_head_kernel = pl.pallas_call(
    _head_body,
    out_shape=(
        jax.ShapeDtypeStruct((G, OUT), jnp.float32),
        jax.ShapeDtypeStruct((G, EMB), jnp.float32),
    ),
)


def kernel(x, edge_index, edge_type, batch,
           W1_rel, W1_root, b1, g1, be1,
           W2_rel, W2_root, b2, g2, be2,
           W3_rel, W3_root, b3, g3, be3,
           Wf1, bf1, gf, bef, Wf2, bf2, Wd1, bd1, Wd2, bd2):
    src = edge_index[0]
    dst = edge_index[1]
    pad = E_PAD - E
    src2d = jnp.pad(src, (0, pad)).reshape(NROWS, CH)
    dst2d = jnp.pad(dst, (0, pad)).reshape(NROWS, CH)
    et2d = jnp.pad(edge_type, (0, pad), constant_values=15).reshape(NROWS, CH)
    zc = jnp.zeros((N, 16), jnp.float32)
    za = jnp.zeros((N, H), jnp.float32)

    cnt_part = _cnt_kernel(dst2d, et2d, zc)
    inv2, offs, cntg = _prep_tc(cnt_part, batch)
    w2d = _prep_w(dst2d, et2d, inv2.reshape(-1))

    def layer(h_kernel, xprev, W_rel, W_root, b, g, be):
        h, root = h_kernel(xprev[:N], W_rel, W_root, b)
        agg_part = _msg_kernel(h.reshape(R * N, H), src2d, et2d, w2d, dst2d, za)
        return _bn_kernel(agg_part, root, g, be)

    x1 = layer(_h_kernel_1, x, W1_rel, W1_root, b1, g1, be1)
    x2 = layer(_h_kernel_2, x1, W2_rel, W2_root, b2, g2, be2)
    x3 = layer(_h_kernel_2, x2, W3_rel, W3_root, b3, g3, be3)

    pooled = _pool_kernel(x1, x2, x3, offs, cntg)
    pred, emb = _head_kernel(pooled, Wf1, bf1, gf, bef, Wf2, bf2,
                             Wd1, bd1, Wd2, bd2)
    return (pred, emb)
